# Initial kernel scaffold; baseline (speedup 1.0000x reference)
import jax
import jax.numpy as jnp
from jax.experimental import pallas as pl


def kernel(x, adj_matrix, W1, b1, W2, b2, W3, b3, Wl, bl):
    raise NotImplementedError("write your pallas kernel here")


# speedup vs baseline: 10.7032x; 10.7032x over previous
import functools

import jax
import jax.numpy as jnp
from jax import lax
from jax.experimental import pallas as pl
from jax.experimental.pallas import tpu as pltpu
from jax.experimental.pallas import tpu_sc as plsc

N = 10000
D = 128
E = 320000
NC = 2
NS = 16
K = 128
CH = 79
TILE_E = CH * K
SLOTS = NC * NS * TILE_E
NPAD = 10240
RPT = NPAD // NS
PAD_NODE = N

@functools.cache
def _sc_mesh():
    return plsc.VectorSubcoreMesh(core_axis_name="c", subcore_axis_name="s",
                                  num_cores=NC, num_subcores=NS)


def _sc_degree_body(dst_hbm, deg_hbm, dst_v, ones_v, zero_v, acc, *, nchunks):
    c = lax.axis_index("c")
    s = lax.axis_index("s")
    pltpu.sync_copy(dst_hbm.at[c, s], dst_v)
    for i in range(K):
        zero_v[i, :] = jnp.zeros((16,), jnp.float32)
        ones_v[i, :] = jnp.ones((16,), jnp.float32)
    base = s * RPT
    for j in range(RPT // K):
        pltpu.sync_copy(zero_v, acc.at[pl.ds(base + j * K, K)])
    plsc.subcore_barrier()

    @pl.loop(0, nchunks)
    def _chunk(j):
        pltpu.sync_copy(ones_v, acc.at[dst_v.at[j]], add=True)

    plsc.subcore_barrier()
    pltpu.sync_copy(acc.at[pl.ds(base, RPT)], deg_hbm.at[c, pl.ds(base, RPT)])


@jax.jit
def _sc_degree(dst_t):
    body = functools.partial(_sc_degree_body, nchunks=CH)
    f = pl.kernel(
        body,
        out_type=jax.ShapeDtypeStruct((NC, NPAD, 16), jnp.float32),
        mesh=_sc_mesh(),
        scratch_types=[
            pltpu.VMEM((CH, K), jnp.int32),
            pltpu.VMEM((K, 16), jnp.float32),
            pltpu.VMEM((K, 16), jnp.float32),
            pltpu.VMEM_SHARED((NPAD, 16), jnp.float32),
        ],
    )
    return f(dst_t)


def _sc_agg_body(g_hbm, src_hbm, dst_hbm, p_hbm,
                 src_v, dst_v, rows0, acc, sem0):
    c = lax.axis_index("c")
    s = lax.axis_index("s")
    pltpu.sync_copy(src_hbm.at[c, s], src_v)
    pltpu.sync_copy(dst_hbm.at[c, s], dst_v)
    pltpu.sync_copy(g_hbm.at[pl.ds(NPAD - K, K)], rows0)
    base = s * RPT
    for j in range(RPT // K):
        pltpu.sync_copy(rows0, acc.at[pl.ds(base + j * K, K)])
    plsc.subcore_barrier()

    @pl.loop(0, CH)
    def _chunk(j):
        pltpu.async_copy(g_hbm.at[src_v.at[j]], rows0, sem0).wait()
        pltpu.sync_copy(rows0, acc.at[dst_v.at[j]], add=True)

    plsc.subcore_barrier()
    pltpu.sync_copy(acc.at[pl.ds(base, RPT)], p_hbm.at[c, pl.ds(base, RPT)])


@jax.jit
def _sc_agg(g, src_t, dst_t):
    f = pl.kernel(
        _sc_agg_body,
        out_type=jax.ShapeDtypeStruct((NC, NPAD, D), jnp.float32),
        mesh=_sc_mesh(),
        scratch_types=[
            pltpu.VMEM((CH, K), jnp.int32),
            pltpu.VMEM((CH, K), jnp.int32),
            pltpu.VMEM((K, D), jnp.float32),
            pltpu.VMEM_SHARED((NPAD, D), jnp.float32),
            pltpu.SemaphoreType.DMA,
        ],
    )
    return f(g, src_t, dst_t)


BLK = 2048


def _tc_prep_body(x_ref, degp_ref, g_ref, dinv_ref):
    i = pl.program_id(0)
    deg = degp_ref[0, :, 0:1] + degp_ref[1, :, 0:1] + 1.0
    dinv = lax.rsqrt(deg)
    rows = i * BLK + lax.broadcasted_iota(jnp.int32, (BLK, 1), 0)
    dinv = jnp.where(rows < N, dinv, 0.0)
    g_ref[...] = x_ref[...] * dinv
    dinv_ref[...] = jnp.broadcast_to(dinv, (BLK, 16))


@jax.jit
def _tc_prep(x_pad, degp):
    return pl.pallas_call(
        _tc_prep_body,
        grid=(NPAD // BLK,),
        in_specs=[
            pl.BlockSpec((BLK, D), lambda i: (i, 0)),
            pl.BlockSpec((NC, BLK, 16), lambda i: (0, i, 0)),
        ],
        out_specs=[
            pl.BlockSpec((BLK, D), lambda i: (i, 0)),
            pl.BlockSpec((BLK, 16), lambda i: (i, 0)),
        ],
        out_shape=[
            jax.ShapeDtypeStruct((NPAD, D), jnp.float32),
            jax.ShapeDtypeStruct((NPAD, 16), jnp.float32),
        ],
    )(x_pad, degp)


def _tc_layer_body(p_ref, g_ref, dinv_ref, w_ref, b_ref, out_ref):
    dinv = dinv_ref[:, 0:1]
    t = (p_ref[0] + p_ref[1] + g_ref[...]) * dinv
    h = jnp.dot(t, w_ref[...], preferred_element_type=jnp.float32) + b_ref[...]
    out_ref[...] = jnp.maximum(h, 0.0) * dinv


@jax.jit
def _tc_layer(p, g, dinv16, W, b):
    return pl.pallas_call(
        _tc_layer_body,
        grid=(NPAD // BLK,),
        in_specs=[
            pl.BlockSpec((NC, BLK, D), lambda i: (0, i, 0)),
            pl.BlockSpec((BLK, D), lambda i: (i, 0)),
            pl.BlockSpec((BLK, 16), lambda i: (i, 0)),
            pl.BlockSpec((D, D), lambda i: (0, 0)),
            pl.BlockSpec((1, D), lambda i: (0, 0)),
        ],
        out_specs=pl.BlockSpec((BLK, D), lambda i: (i, 0)),
        out_shape=jax.ShapeDtypeStruct((NPAD, D), jnp.float32),
    )(p, g, dinv16, W, b)


FBLK = 2000


def _tc_final_body(p_ref, g_ref, dinv_ref, w3_ref, b3_ref, wl_ref, bl_ref,
                   out_ref):
    dinv = dinv_ref[:, 0:1]
    t = (p_ref[0] + p_ref[1] + g_ref[...]) * dinv
    h = jnp.dot(t, w3_ref[...], preferred_element_type=jnp.float32)
    h = jnp.maximum(h + b3_ref[...], 0.0)
    z = jnp.dot(h, wl_ref[...], preferred_element_type=jnp.float32)
    z = z + bl_ref[...]
    z = z - jnp.max(z, axis=1, keepdims=True)
    ez = jnp.exp(z)
    out_ref[...] = ez / jnp.sum(ez, axis=1, keepdims=True)


@jax.jit
def _tc_final(p, g, dinv16, W3, b3, Wl, bl):
    return pl.pallas_call(
        _tc_final_body,
        grid=(N // FBLK,),
        in_specs=[
            pl.BlockSpec((NC, FBLK, D), lambda i: (0, i, 0)),
            pl.BlockSpec((FBLK, D), lambda i: (i, 0)),
            pl.BlockSpec((FBLK, 16), lambda i: (i, 0)),
            pl.BlockSpec((D, D), lambda i: (0, 0)),
            pl.BlockSpec((1, D), lambda i: (0, 0)),
            pl.BlockSpec((D, D), lambda i: (0, 0)),
            pl.BlockSpec((1, D), lambda i: (0, 0)),
        ],
        out_specs=pl.BlockSpec((FBLK, D), lambda i: (i, 0)),
        out_shape=jax.ShapeDtypeStruct((N, D), jnp.float32),
    )(p, g, dinv16, W3, b3, Wl, bl)


def kernel(x, adj_matrix, W1, b1, W2, b2, W3, b3, Wl, bl):
    src = adj_matrix[0].astype(jnp.int32)
    dst = adj_matrix[1].astype(jnp.int32)
    pad = jnp.full((SLOTS - E,), PAD_NODE, jnp.int32)
    src_t = jnp.concatenate([src, pad]).reshape(NC, NS, CH, K)
    dst_t = jnp.concatenate([dst, pad]).reshape(NC, NS, CH, K)
    x_pad = jnp.zeros((NPAD, D), jnp.float32).at[:N].set(x)

    degp = _sc_degree(dst_t)
    g, dinv16 = _tc_prep(x_pad, degp)
    for (W, b) in ((W1, b1), (W2, b2)):
        p = _sc_agg(g, src_t, dst_t)
        g = _tc_layer(p, g, dinv16, W, b.reshape(1, D))
    p = _sc_agg(g, src_t, dst_t)
    return _tc_final(p, g, dinv16, W3, b3.reshape(1, D), Wl, bl.reshape(1, D))
